# Initial kernel scaffold; baseline (speedup 1.0000x reference)
import functools

import jax
import jax.numpy as jnp
from jax import lax
from jax.experimental import pallas as pl
from jax.experimental.pallas import tpu as pltpu
from jax.experimental.pallas import tpu_sc as plsc

N = 10000
E = 160000
NG = 64
CI = 16
CEH = 32
CW = 256

NT = 32
CPT_R = E // NT
CHK = 128
NCHK = 40
CPT = NCHK * CHK
EPAD = NT * CPT
NPAD = 10016
DUMMY = N
NSUB = 16
RPS = NPAD // NSUB

_mesh = plsc.VectorSubcoreMesh(core_axis_name="c", subcore_axis_name="s")


@functools.partial(
    pl.kernel,
    out_type=jax.ShapeDtypeStruct((NT, NCHK, CHK, CI), jnp.float32),
    mesh=_mesh,
    scratch_types=[
        pltpu.VMEM((NCHK, CHK), jnp.int32),
        pltpu.VMEM((NCHK, CHK, CI), jnp.float32),
        pltpu.SemaphoreType.DMA,
    ],
)
def _sc_gather(table_hbm, idx_hbm, out_hbm, idx_v, rows_v, sem):
    wid = lax.axis_index("s") * 2 + lax.axis_index("c")
    pltpu.sync_copy(idx_hbm.at[wid], idx_v)

    def body(k, carry):
        pltpu.async_copy(table_hbm.at[idx_v.at[k]], rows_v.at[k], sem).wait()
        return carry

    lax.fori_loop(0, NCHK, body, 0)
    pltpu.sync_copy(rows_v, out_hbm.at[wid])


@functools.partial(
    pl.kernel,
    out_type=jax.ShapeDtypeStruct((2, NPAD, CI), jnp.float32),
    mesh=_mesh,
    scratch_types=[
        pltpu.VMEM((NCHK, CHK), jnp.int32),
        pltpu.VMEM((NCHK, CHK, CI), jnp.float32),
        pltpu.VMEM_SHARED((NPAD, CI), jnp.float32),
    ],
)
def _sc_scatter(msg_hbm, idx_hbm, zeros_hbm, out_hbm, idx_v, rows_v, acc_sh):
    c = lax.axis_index("c")
    s = lax.axis_index("s")
    wid = s * 2 + c
    pltpu.sync_copy(zeros_hbm.at[pl.ds(s * RPS, RPS)],
                    acc_sh.at[pl.ds(s * RPS, RPS)])
    pltpu.sync_copy(idx_hbm.at[wid], idx_v)
    pltpu.sync_copy(msg_hbm.at[wid], rows_v)
    plsc.subcore_barrier()

    def body(k, carry):
        pltpu.sync_copy(rows_v.at[k], acc_sh.at[idx_v.at[k]], add=True)
        return carry

    lax.fori_loop(0, NCHK, body, 0)
    plsc.subcore_barrier()
    pltpu.sync_copy(acc_sh.at[pl.ds(s * RPS, RPS)],
                    out_hbm.at[c, pl.ds(s * RPS, RPS)])


BLK = 2048


def _edge_body(ea_ref, xs_ref, w1_ref, b1_ref, w2_ref, b2_ref, r_ref, s_ref,
               o_ref):
    f32 = jnp.float32
    h = jnp.maximum(
        jnp.dot(ea_ref[...], w1_ref[...], preferred_element_type=f32)
        + b1_ref[...], 0.0)
    w = jnp.maximum(
        jnp.dot(h, w2_ref[...], preferred_element_type=f32) + b2_ref[...],
        0.0)
    xe = jnp.dot(xs_ref[...], r_ref[...], preferred_element_type=f32)
    o_ref[...] = jnp.dot(xe * w, s_ref[...], preferred_element_type=f32)


def _tc_edge(ea, xs, W1, b1, W2, b2, Rm, Sm):
    return pl.pallas_call(
        _edge_body,
        grid=(EPAD // BLK,),
        in_specs=[
            pl.BlockSpec((BLK, CI), lambda i: (i, 0)),
            pl.BlockSpec((BLK, CI), lambda i: (i, 0)),
            pl.BlockSpec((CI, CEH), lambda i: (0, 0)),
            pl.BlockSpec((1, CEH), lambda i: (0, 0)),
            pl.BlockSpec((CEH, CW), lambda i: (0, 0)),
            pl.BlockSpec((1, CW), lambda i: (0, 0)),
            pl.BlockSpec((CI, CW), lambda i: (0, 0)),
            pl.BlockSpec((CW, CI), lambda i: (0, 0)),
        ],
        out_specs=pl.BlockSpec((BLK, CI), lambda i: (i, 0)),
        out_shape=jax.ShapeDtypeStruct((EPAD, CI), jnp.float32),
    )(ea, xs, W1, b1, W2, b2, Rm, Sm)


def _node_body(aggp_ref, x_ref, root_ref, bias_ref, o_ref):
    h = (aggp_ref[0] + aggp_ref[1]
         + jnp.dot(x_ref[...], root_ref[...],
                   preferred_element_type=jnp.float32) + bias_ref[...])
    o_ref[...] = jnp.maximum(h, 0.0)


def _tc_node(aggp, xpad, root, bias):
    return pl.pallas_call(
        _node_body,
        out_shape=jax.ShapeDtypeStruct((NPAD, CI), jnp.float32),
    )(aggp, xpad, root, bias.reshape(1, CI))


def _final_body(aggp_ref, h_ref, root_ref, bias_ref, bi_ref, w1_ref, c1_ref,
                w2_ref, c2_ref, w3_ref, c3_ref, hw_ref, hb_ref, o_ref):
    f32 = jnp.float32
    h2 = (aggp_ref[0] + aggp_ref[1]
          + jnp.dot(h_ref[...], root_ref[...], preferred_element_type=f32)
          + bias_ref[...])
    ohT = (bi_ref[...] == lax.broadcasted_iota(jnp.int32, (NG, NPAD), 0)
           ).astype(f32)
    sums = jnp.dot(ohT, h2, preferred_element_type=f32)
    cnts = jnp.dot(ohT, jnp.ones_like(h2), preferred_element_type=f32)
    g = sums / jnp.maximum(cnts, 1.0)
    g = jnp.maximum(jnp.dot(g, w1_ref[...], preferred_element_type=f32)
                    + c1_ref[...], 0.0)
    g = jnp.maximum(jnp.dot(g, w2_ref[...], preferred_element_type=f32)
                    + c2_ref[...], 0.0)
    g = jnp.maximum(jnp.dot(g, w3_ref[...], preferred_element_type=f32)
                    + c3_ref[...], 0.0)
    o_ref[...] = (jnp.dot(g, hw_ref[...], preferred_element_type=f32)
                  + hb_ref[...])


def _tc_final(aggp, h, root, bias, bip, w1, c1, w2, c2, w3, c3, hw, hb):
    return pl.pallas_call(
        _final_body,
        out_shape=jax.ShapeDtypeStruct((NG, 8), jnp.float32),
    )(aggp, h, root, bias.reshape(1, CI), bip,
      w1, c1.reshape(1, CI), w2, c2.reshape(1, CI), w3, c3.reshape(1, CI),
      hw, hb.reshape(1, 8))


def kernel(x, edge_index, edge_attr, batch_idx,
           e1_W1, e1_b1, e1_W2, e1_b2, root1, bias1,
           e2_W1, e2_b1, e2_W2, e2_b2, root2, bias2,
           nn_W1, nn_b1, nn_W2, nn_b2, nn_W3, nn_b3,
           head_W, head_b):
    f32 = jnp.float32
    pad = CPT - CPT_R
    src = edge_index[0].reshape(NT, CPT_R)
    dst = edge_index[1].reshape(NT, CPT_R)
    srcp = jnp.pad(src, ((0, 0), (0, pad)),
                   constant_values=DUMMY).reshape(NT, NCHK, CHK)
    dstp = jnp.pad(dst, ((0, 0), (0, pad)),
                   constant_values=DUMMY).reshape(NT, NCHK, CHK)
    eap = jnp.pad(edge_attr.reshape(NT, CPT_R, CI),
                  ((0, 0), (0, pad), (0, 0))).reshape(EPAD, CI)
    xpad = jnp.pad(x, ((0, NPAD - N), (0, 0)))
    bip = jnp.pad(batch_idx, (0, NPAD - N),
                  constant_values=NG).reshape(1, NPAD)
    zN = jnp.zeros((NPAD, CI), f32)
    jj = jnp.arange(CW)
    Rm = (jj[None, :] // CI == jnp.arange(CI)[:, None]).astype(f32)
    Sm = (jj[:, None] % CI == jnp.arange(CI)[None, :]).astype(f32)

    xs = _sc_gather(xpad, srcp).reshape(EPAD, CI)
    msg1 = _tc_edge(eap, xs, e1_W1, e1_b1.reshape(1, CEH), e1_W2,
                    e1_b2.reshape(1, CW), Rm, Sm)
    agg1 = _sc_scatter(msg1.reshape(NT, NCHK, CHK, CI), dstp, zN)
    h = _tc_node(agg1, xpad, root1, bias1)
    hs = _sc_gather(h, srcp).reshape(EPAD, CI)
    msg2 = _tc_edge(eap, hs, e2_W1, e2_b1.reshape(1, CEH), e2_W2,
                    e2_b2.reshape(1, CW), Rm, Sm)
    agg2 = _sc_scatter(msg2.reshape(NT, NCHK, CHK, CI), dstp, zN)
    return _tc_final(agg2, h, root2, bias2, bip,
                     nn_W1, nn_b1, nn_W2, nn_b2, nn_W3, nn_b3,
                     head_W, head_b)


# speedup vs baseline: 3.0976x; 3.0976x over previous
import functools

import jax
import jax.numpy as jnp
from jax import lax
from jax.experimental import pallas as pl
from jax.experimental.pallas import tpu as pltpu
from jax.experimental.pallas import tpu_sc as plsc

N = 10000
E = 160000
NG = 64
CI = 16
CEH = 32
CW = 256

NT = 32
CPT_R = E // NT
CHK = 128
NCHK = 40
CPT = NCHK * CHK
EPAD = NT * CPT
NPAD = 10016
DUMMY = N
NSUB = 16
RPS = NPAD // NSUB

@functools.lru_cache(maxsize=None)
def _sc_kernels():
    mesh = plsc.VectorSubcoreMesh(core_axis_name="c", subcore_axis_name="s")

    @functools.partial(
        pl.kernel,
        out_type=jax.ShapeDtypeStruct((NT, NCHK, CHK, CI), jnp.float32),
        mesh=mesh,
        scratch_types=[
            pltpu.VMEM((NCHK, CHK), jnp.int32),
            pltpu.VMEM((NCHK, CHK, CI), jnp.float32),
            pltpu.SemaphoreType.DMA,
        ],
        compiler_params=pltpu.CompilerParams(use_tc_tiling_on_sc=False),
    )
    def _sc_gather(table_hbm, idx_hbm, out_hbm, idx_v, rows_v, sem):
        wid = lax.axis_index("s") * 2 + lax.axis_index("c")
        pltpu.sync_copy(idx_hbm.at[wid], idx_v)

        def body(k, carry):
            pltpu.async_copy(table_hbm.at[idx_v.at[k]], rows_v.at[k],
                             sem).wait()
            return carry

        lax.fori_loop(0, NCHK, body, 0)
        pltpu.sync_copy(rows_v, out_hbm.at[wid])

    @functools.partial(
        pl.kernel,
        out_type=jax.ShapeDtypeStruct((2, NPAD, CI), jnp.float32),
        mesh=mesh,
        scratch_types=[
            pltpu.VMEM((NCHK, CHK), jnp.int32),
            pltpu.VMEM((NCHK, CHK, CI), jnp.float32),
            pltpu.VMEM_SHARED((NPAD, CI), jnp.float32),
        ],
        compiler_params=pltpu.CompilerParams(use_tc_tiling_on_sc=False),
    )
    def _sc_scatter(msg_hbm, idx_hbm, zeros_hbm, out_hbm, idx_v, rows_v,
                    acc_sh):
        c = lax.axis_index("c")
        s = lax.axis_index("s")
        wid = s * 2 + c
        pltpu.sync_copy(zeros_hbm.at[pl.ds(s * RPS, RPS)],
                        acc_sh.at[pl.ds(s * RPS, RPS)])
        pltpu.sync_copy(idx_hbm.at[wid], idx_v)
        pltpu.sync_copy(msg_hbm.at[wid], rows_v)
        plsc.subcore_barrier()

        def body(k, carry):
            pltpu.sync_copy(rows_v.at[k], acc_sh.at[idx_v.at[k]], add=True)
            return carry

        lax.fori_loop(0, NCHK, body, 0)
        plsc.subcore_barrier()
        pltpu.sync_copy(acc_sh.at[pl.ds(s * RPS, RPS)],
                        out_hbm.at[c, pl.ds(s * RPS, RPS)])

    return _sc_gather, _sc_scatter


BLK = 2048


def _edge_body(ea_ref, xs_ref, w1_ref, b1_ref, w2_ref, b2_ref, r_ref, s_ref,
               o_ref):
    f32 = jnp.float32
    h = jnp.maximum(
        jnp.dot(ea_ref[...], w1_ref[...], preferred_element_type=f32)
        + b1_ref[...], 0.0)
    w = jnp.maximum(
        jnp.dot(h, w2_ref[...], preferred_element_type=f32) + b2_ref[...],
        0.0)
    xe = jnp.dot(xs_ref[...], r_ref[...], preferred_element_type=f32)
    o_ref[...] = jnp.dot(xe * w, s_ref[...], preferred_element_type=f32)


def _tc_edge(ea, xs, W1, b1, W2, b2, Rm, Sm):
    return pl.pallas_call(
        _edge_body,
        grid=(EPAD // BLK,),
        in_specs=[
            pl.BlockSpec((BLK, CI), lambda i: (i, 0)),
            pl.BlockSpec((BLK, CI), lambda i: (i, 0)),
            pl.BlockSpec((CI, CEH), lambda i: (0, 0)),
            pl.BlockSpec((1, CEH), lambda i: (0, 0)),
            pl.BlockSpec((CEH, CW), lambda i: (0, 0)),
            pl.BlockSpec((1, CW), lambda i: (0, 0)),
            pl.BlockSpec((CI, CW), lambda i: (0, 0)),
            pl.BlockSpec((CW, CI), lambda i: (0, 0)),
        ],
        out_specs=pl.BlockSpec((BLK, CI), lambda i: (i, 0)),
        out_shape=jax.ShapeDtypeStruct((EPAD, CI), jnp.float32),
    )(ea, xs, W1, b1, W2, b2, Rm, Sm)


def _node_body(aggp_ref, x_ref, root_ref, bias_ref, o_ref):
    h = (aggp_ref[0] + aggp_ref[1]
         + jnp.dot(x_ref[...], root_ref[...],
                   preferred_element_type=jnp.float32) + bias_ref[...])
    o_ref[...] = jnp.maximum(h, 0.0)


def _tc_node(aggp, xpad, root, bias):
    return pl.pallas_call(
        _node_body,
        out_shape=jax.ShapeDtypeStruct((NPAD, CI), jnp.float32),
    )(aggp, xpad, root, bias.reshape(1, CI))


def _final_body(aggp_ref, h_ref, root_ref, bias_ref, bi_ref, w1_ref, c1_ref,
                w2_ref, c2_ref, w3_ref, c3_ref, hw_ref, hb_ref, o_ref):
    f32 = jnp.float32
    h2 = (aggp_ref[0] + aggp_ref[1]
          + jnp.dot(h_ref[...], root_ref[...], preferred_element_type=f32)
          + bias_ref[...])
    ohT = (bi_ref[...] == lax.broadcasted_iota(jnp.int32, (NG, NPAD), 0)
           ).astype(f32)
    sums = jnp.dot(ohT, h2, preferred_element_type=f32)
    cnts = jnp.dot(ohT, jnp.ones_like(h2), preferred_element_type=f32)
    g = sums / jnp.maximum(cnts, 1.0)
    g = jnp.maximum(jnp.dot(g, w1_ref[...], preferred_element_type=f32)
                    + c1_ref[...], 0.0)
    g = jnp.maximum(jnp.dot(g, w2_ref[...], preferred_element_type=f32)
                    + c2_ref[...], 0.0)
    g = jnp.maximum(jnp.dot(g, w3_ref[...], preferred_element_type=f32)
                    + c3_ref[...], 0.0)
    o_ref[...] = (jnp.dot(g, hw_ref[...], preferred_element_type=f32)
                  + hb_ref[...])


def _tc_final(aggp, h, root, bias, bip, w1, c1, w2, c2, w3, c3, hw, hb):
    return pl.pallas_call(
        _final_body,
        out_shape=jax.ShapeDtypeStruct((NG, 8), jnp.float32),
    )(aggp, h, root, bias.reshape(1, CI), bip,
      w1, c1.reshape(1, CI), w2, c2.reshape(1, CI), w3, c3.reshape(1, CI),
      hw, hb.reshape(1, 8))


def kernel(x, edge_index, edge_attr, batch_idx,
           e1_W1, e1_b1, e1_W2, e1_b2, root1, bias1,
           e2_W1, e2_b1, e2_W2, e2_b2, root2, bias2,
           nn_W1, nn_b1, nn_W2, nn_b2, nn_W3, nn_b3,
           head_W, head_b):
    f32 = jnp.float32
    pad = CPT - CPT_R
    src = edge_index[0].reshape(NT, CPT_R)
    dst = edge_index[1].reshape(NT, CPT_R)
    srcp = jnp.pad(src, ((0, 0), (0, pad)),
                   constant_values=DUMMY).reshape(NT, NCHK, CHK)
    dstp = jnp.pad(dst, ((0, 0), (0, pad)),
                   constant_values=DUMMY).reshape(NT, NCHK, CHK)
    eap = jnp.pad(edge_attr.reshape(NT, CPT_R, CI),
                  ((0, 0), (0, pad), (0, 0))).reshape(EPAD, CI)
    xpad = jnp.pad(x, ((0, NPAD - N), (0, 0)))
    bip = jnp.pad(batch_idx, (0, NPAD - N),
                  constant_values=NG).reshape(1, NPAD)
    zN = jnp.zeros((NPAD, CI), f32)
    jj = jnp.arange(CW)
    Rm = (jj[None, :] // CI == jnp.arange(CI)[:, None]).astype(f32)
    Sm = (jj[:, None] % CI == jnp.arange(CI)[None, :]).astype(f32)

    sc_gather, sc_scatter = _sc_kernels()
    xs = sc_gather(xpad, srcp).reshape(EPAD, CI)
    msg1 = _tc_edge(eap, xs, e1_W1, e1_b1.reshape(1, CEH), e1_W2,
                    e1_b2.reshape(1, CW), Rm, Sm)
    agg1 = sc_scatter(msg1.reshape(NT, NCHK, CHK, CI), dstp, zN)
    h = _tc_node(agg1, xpad, root1, bias1)
    hs = sc_gather(h, srcp).reshape(EPAD, CI)
    msg2 = _tc_edge(eap, hs, e2_W1, e2_b1.reshape(1, CEH), e2_W2,
                    e2_b2.reshape(1, CW), Rm, Sm)
    agg2 = sc_scatter(msg2.reshape(NT, NCHK, CHK, CI), dstp, zN)
    return _tc_final(agg2, h, root2, bias2, bip,
                     nn_W1, nn_b1, nn_W2, nn_b2, nn_W3, nn_b3,
                     head_W, head_b)


# speedup vs baseline: 3.0980x; 1.0001x over previous
import functools

import jax
import jax.numpy as jnp
from jax import lax
from jax.experimental import pallas as pl
from jax.experimental.pallas import tpu as pltpu
from jax.experimental.pallas import tpu_sc as plsc

N = 10000
E = 160000
NG = 64
CI = 16
CEH = 32
CW = 256

NT = 32
CPT_R = E // NT
CHK = 128
NCHK = 40
CPT = NCHK * CHK
EPAD = NT * CPT
NPAD = 10016
DUMMY = N
NSUB = 16
RPS = NPAD // NSUB

@functools.lru_cache(maxsize=None)
def _sc_kernels():
    mesh = plsc.VectorSubcoreMesh(core_axis_name="c", subcore_axis_name="s")

    @functools.partial(
        pl.kernel,
        out_type=jax.ShapeDtypeStruct((NT, NCHK, CHK, CI), jnp.float32),
        mesh=mesh,
        scratch_types=[
            pltpu.VMEM((NCHK, CHK), jnp.int32),
            pltpu.VMEM((NCHK, CHK, CI), jnp.float32),
            pltpu.SemaphoreType.DMA,
        ],
        compiler_params=pltpu.CompilerParams(use_tc_tiling_on_sc=False),
        name="sc_gather",
    )
    def _sc_gather(table_hbm, idx_hbm, out_hbm, idx_v, rows_v, sem):
        wid = lax.axis_index("s") * 2 + lax.axis_index("c")
        pltpu.sync_copy(idx_hbm.at[wid], idx_v)

        def body(k, carry):
            pltpu.async_copy(table_hbm.at[idx_v.at[k]], rows_v.at[k],
                             sem).wait()
            return carry

        lax.fori_loop(0, NCHK, body, 0)
        pltpu.sync_copy(rows_v, out_hbm.at[wid])

    @functools.partial(
        pl.kernel,
        out_type=jax.ShapeDtypeStruct((2, NPAD, CI), jnp.float32),
        mesh=mesh,
        scratch_types=[
            pltpu.VMEM((NCHK, CHK), jnp.int32),
            pltpu.VMEM((NCHK, CHK, CI), jnp.float32),
            pltpu.VMEM_SHARED((NPAD, CI), jnp.float32),
        ],
        compiler_params=pltpu.CompilerParams(use_tc_tiling_on_sc=False),
        name="sc_scatter",
    )
    def _sc_scatter(msg_hbm, idx_hbm, zeros_hbm, out_hbm, idx_v, rows_v,
                    acc_sh):
        c = lax.axis_index("c")
        s = lax.axis_index("s")
        wid = s * 2 + c
        pltpu.sync_copy(zeros_hbm.at[pl.ds(s * RPS, RPS)],
                        acc_sh.at[pl.ds(s * RPS, RPS)])
        pltpu.sync_copy(idx_hbm.at[wid], idx_v)
        pltpu.sync_copy(msg_hbm.at[wid], rows_v)
        plsc.subcore_barrier()

        def body(k, carry):
            pltpu.sync_copy(rows_v.at[k], acc_sh.at[idx_v.at[k]], add=True)
            return carry

        lax.fori_loop(0, NCHK, body, 0)
        plsc.subcore_barrier()
        pltpu.sync_copy(acc_sh.at[pl.ds(s * RPS, RPS)],
                        out_hbm.at[c, pl.ds(s * RPS, RPS)])

    return _sc_gather, _sc_scatter


BLK = 2048


def _edge_body(ea_ref, xs_ref, w1_ref, b1_ref, w2_ref, b2_ref, r_ref, s_ref,
               o_ref):
    f32 = jnp.float32
    h = jnp.maximum(
        jnp.dot(ea_ref[...], w1_ref[...], preferred_element_type=f32)
        + b1_ref[...], 0.0)
    w = jnp.maximum(
        jnp.dot(h, w2_ref[...], preferred_element_type=f32) + b2_ref[...],
        0.0)
    xe = jnp.dot(xs_ref[...], r_ref[...], preferred_element_type=f32)
    o_ref[...] = jnp.dot(xe * w, s_ref[...], preferred_element_type=f32)


def _tc_edge(ea, xs, W1, b1, W2, b2, Rm, Sm):
    return pl.pallas_call(
        _edge_body,
        grid=(EPAD // BLK,),
        in_specs=[
            pl.BlockSpec((BLK, CI), lambda i: (i, 0)),
            pl.BlockSpec((BLK, CI), lambda i: (i, 0)),
            pl.BlockSpec((CI, CEH), lambda i: (0, 0)),
            pl.BlockSpec((1, CEH), lambda i: (0, 0)),
            pl.BlockSpec((CEH, CW), lambda i: (0, 0)),
            pl.BlockSpec((1, CW), lambda i: (0, 0)),
            pl.BlockSpec((CI, CW), lambda i: (0, 0)),
            pl.BlockSpec((CW, CI), lambda i: (0, 0)),
        ],
        out_specs=pl.BlockSpec((BLK, CI), lambda i: (i, 0)),
        out_shape=jax.ShapeDtypeStruct((EPAD, CI), jnp.float32),
        name="tc_edge",
    )(ea, xs, W1, b1, W2, b2, Rm, Sm)


def _node_body(aggp_ref, x_ref, root_ref, bias_ref, o_ref):
    h = (aggp_ref[0] + aggp_ref[1]
         + jnp.dot(x_ref[...], root_ref[...],
                   preferred_element_type=jnp.float32) + bias_ref[...])
    o_ref[...] = jnp.maximum(h, 0.0)


def _tc_node(aggp, xpad, root, bias):
    return pl.pallas_call(
        _node_body,
        out_shape=jax.ShapeDtypeStruct((NPAD, CI), jnp.float32),
        name="tc_node",
    )(aggp, xpad, root, bias.reshape(1, CI))


def _final_body(aggp_ref, h_ref, root_ref, bias_ref, bi_ref, w1_ref, c1_ref,
                w2_ref, c2_ref, w3_ref, c3_ref, hw_ref, hb_ref, o_ref):
    f32 = jnp.float32
    h2 = (aggp_ref[0] + aggp_ref[1]
          + jnp.dot(h_ref[...], root_ref[...], preferred_element_type=f32)
          + bias_ref[...])
    ohT = (bi_ref[...] == lax.broadcasted_iota(jnp.int32, (NG, NPAD), 0)
           ).astype(f32)
    sums = jnp.dot(ohT, h2, preferred_element_type=f32)
    cnts = jnp.dot(ohT, jnp.ones_like(h2), preferred_element_type=f32)
    g = sums / jnp.maximum(cnts, 1.0)
    g = jnp.maximum(jnp.dot(g, w1_ref[...], preferred_element_type=f32)
                    + c1_ref[...], 0.0)
    g = jnp.maximum(jnp.dot(g, w2_ref[...], preferred_element_type=f32)
                    + c2_ref[...], 0.0)
    g = jnp.maximum(jnp.dot(g, w3_ref[...], preferred_element_type=f32)
                    + c3_ref[...], 0.0)
    o_ref[...] = (jnp.dot(g, hw_ref[...], preferred_element_type=f32)
                  + hb_ref[...])


def _tc_final(aggp, h, root, bias, bip, w1, c1, w2, c2, w3, c3, hw, hb):
    return pl.pallas_call(
        _final_body,
        out_shape=jax.ShapeDtypeStruct((NG, 8), jnp.float32),
        name="tc_final",
    )(aggp, h, root, bias.reshape(1, CI), bip,
      w1, c1.reshape(1, CI), w2, c2.reshape(1, CI), w3, c3.reshape(1, CI),
      hw, hb.reshape(1, 8))


def kernel(x, edge_index, edge_attr, batch_idx,
           e1_W1, e1_b1, e1_W2, e1_b2, root1, bias1,
           e2_W1, e2_b1, e2_W2, e2_b2, root2, bias2,
           nn_W1, nn_b1, nn_W2, nn_b2, nn_W3, nn_b3,
           head_W, head_b):
    f32 = jnp.float32
    pad = CPT - CPT_R
    src = edge_index[0].reshape(NT, CPT_R)
    dst = edge_index[1].reshape(NT, CPT_R)
    srcp = jnp.pad(src, ((0, 0), (0, pad)),
                   constant_values=DUMMY).reshape(NT, NCHK, CHK)
    dstp = jnp.pad(dst, ((0, 0), (0, pad)),
                   constant_values=DUMMY).reshape(NT, NCHK, CHK)
    eap = jnp.pad(edge_attr.reshape(NT, CPT_R, CI),
                  ((0, 0), (0, pad), (0, 0))).reshape(EPAD, CI)
    xpad = jnp.pad(x, ((0, NPAD - N), (0, 0)))
    bip = jnp.pad(batch_idx, (0, NPAD - N),
                  constant_values=NG).reshape(1, NPAD)
    zN = jnp.zeros((NPAD, CI), f32)
    jj = jnp.arange(CW)
    Rm = (jj[None, :] // CI == jnp.arange(CI)[:, None]).astype(f32)
    Sm = (jj[:, None] % CI == jnp.arange(CI)[None, :]).astype(f32)

    sc_gather, sc_scatter = _sc_kernels()
    xs = sc_gather(xpad, srcp).reshape(EPAD, CI)
    msg1 = _tc_edge(eap, xs, e1_W1, e1_b1.reshape(1, CEH), e1_W2,
                    e1_b2.reshape(1, CW), Rm, Sm)
    agg1 = sc_scatter(msg1.reshape(NT, NCHK, CHK, CI), dstp, zN)
    h = _tc_node(agg1, xpad, root1, bias1)
    hs = sc_gather(h, srcp).reshape(EPAD, CI)
    msg2 = _tc_edge(eap, hs, e2_W1, e2_b1.reshape(1, CEH), e2_W2,
                    e2_b2.reshape(1, CW), Rm, Sm)
    agg2 = sc_scatter(msg2.reshape(NT, NCHK, CHK, CI), dstp, zN)
    return _tc_final(agg2, h, root2, bias2, bip,
                     nn_W1, nn_b1, nn_W2, nn_b2, nn_W3, nn_b3,
                     head_W, head_b)


# speedup vs baseline: 3.4417x; 1.1109x over previous
import functools

import jax
import jax.numpy as jnp
from jax import lax
from jax.experimental import pallas as pl
from jax.experimental.pallas import tpu as pltpu
from jax.experimental.pallas import tpu_sc as plsc

N = 10000
E = 160000
NG = 64
CI = 16
CEH = 32
CW = 256

NT = 32
CPT_R = E // NT
CHK = 128
NCHK = 40
CPT = NCHK * CHK
EPAD = NT * CPT
NPAD = 10016
DUMMY = N
NSUB = 16
RPS = NPAD // NSUB

@functools.lru_cache(maxsize=None)
def _sc_kernels():
    mesh = plsc.VectorSubcoreMesh(core_axis_name="c", subcore_axis_name="s")

    @functools.partial(
        pl.kernel,
        out_type=jax.ShapeDtypeStruct((NT, NCHK, CHK, CI), jnp.float32),
        mesh=mesh,
        scratch_types=[
            pltpu.VMEM((NCHK, CHK), jnp.int32),
            pltpu.VMEM((NCHK, CHK, CI), jnp.float32),
            pltpu.VMEM_SHARED((NPAD, CI), jnp.float32),
        ],
        compiler_params=pltpu.CompilerParams(use_tc_tiling_on_sc=False),
        name="sc_gather",
    )
    def _sc_gather(table_hbm, idx_hbm, out_hbm, idx_v, rows_v, tab_sh):
        s = lax.axis_index("s")
        wid = s * 2 + lax.axis_index("c")
        pltpu.sync_copy(table_hbm.at[pl.ds(s * RPS, RPS)],
                        tab_sh.at[pl.ds(s * RPS, RPS)])
        pltpu.sync_copy(idx_hbm.at[wid], idx_v)
        plsc.subcore_barrier()

        def body(k, carry):
            pltpu.sync_copy(tab_sh.at[idx_v.at[k]], rows_v.at[k])
            return carry

        lax.fori_loop(0, NCHK, body, 0)
        pltpu.sync_copy(rows_v, out_hbm.at[wid])

    @functools.partial(
        pl.kernel,
        out_type=jax.ShapeDtypeStruct((2, NPAD, CI), jnp.float32),
        mesh=mesh,
        scratch_types=[
            pltpu.VMEM((NCHK, CHK), jnp.int32),
            pltpu.VMEM((NCHK, CHK, CI), jnp.float32),
            pltpu.VMEM_SHARED((NPAD, CI), jnp.float32),
        ],
        compiler_params=pltpu.CompilerParams(use_tc_tiling_on_sc=False),
        name="sc_scatter",
    )
    def _sc_scatter(msg_hbm, idx_hbm, zeros_hbm, out_hbm, idx_v, rows_v,
                    acc_sh):
        c = lax.axis_index("c")
        s = lax.axis_index("s")
        wid = s * 2 + c
        pltpu.sync_copy(zeros_hbm.at[pl.ds(s * RPS, RPS)],
                        acc_sh.at[pl.ds(s * RPS, RPS)])
        pltpu.sync_copy(idx_hbm.at[wid], idx_v)
        pltpu.sync_copy(msg_hbm.at[wid], rows_v)
        plsc.subcore_barrier()

        def body(k, carry):
            pltpu.sync_copy(rows_v.at[k], acc_sh.at[idx_v.at[k]], add=True)
            return carry

        lax.fori_loop(0, NCHK, body, 0)
        plsc.subcore_barrier()
        pltpu.sync_copy(acc_sh.at[pl.ds(s * RPS, RPS)],
                        out_hbm.at[c, pl.ds(s * RPS, RPS)])

    return _sc_gather, _sc_scatter


BLK = 2048


def _edge_body(ea_ref, xs_ref, w1_ref, b1_ref, w2_ref, b2_ref, r_ref, s_ref,
               o_ref):
    f32 = jnp.float32
    h = jnp.maximum(
        jnp.dot(ea_ref[...], w1_ref[...], preferred_element_type=f32)
        + b1_ref[...], 0.0)
    w = jnp.maximum(
        jnp.dot(h, w2_ref[...], preferred_element_type=f32) + b2_ref[...],
        0.0)
    xe = jnp.dot(xs_ref[...], r_ref[...], preferred_element_type=f32)
    o_ref[...] = jnp.dot(xe * w, s_ref[...], preferred_element_type=f32)


def _tc_edge(ea, xs, W1, b1, W2, b2, Rm, Sm):
    return pl.pallas_call(
        _edge_body,
        grid=(EPAD // BLK,),
        in_specs=[
            pl.BlockSpec((BLK, CI), lambda i: (i, 0)),
            pl.BlockSpec((BLK, CI), lambda i: (i, 0)),
            pl.BlockSpec((CI, CEH), lambda i: (0, 0)),
            pl.BlockSpec((1, CEH), lambda i: (0, 0)),
            pl.BlockSpec((CEH, CW), lambda i: (0, 0)),
            pl.BlockSpec((1, CW), lambda i: (0, 0)),
            pl.BlockSpec((CI, CW), lambda i: (0, 0)),
            pl.BlockSpec((CW, CI), lambda i: (0, 0)),
        ],
        out_specs=pl.BlockSpec((BLK, CI), lambda i: (i, 0)),
        out_shape=jax.ShapeDtypeStruct((EPAD, CI), jnp.float32),
        name="tc_edge",
    )(ea, xs, W1, b1, W2, b2, Rm, Sm)


def _node_body(aggp_ref, x_ref, root_ref, bias_ref, o_ref):
    h = (aggp_ref[0] + aggp_ref[1]
         + jnp.dot(x_ref[...], root_ref[...],
                   preferred_element_type=jnp.float32) + bias_ref[...])
    o_ref[...] = jnp.maximum(h, 0.0)


def _tc_node(aggp, xpad, root, bias):
    return pl.pallas_call(
        _node_body,
        out_shape=jax.ShapeDtypeStruct((NPAD, CI), jnp.float32),
        name="tc_node",
    )(aggp, xpad, root, bias.reshape(1, CI))


def _final_body(aggp_ref, h_ref, root_ref, bias_ref, bi_ref, w1_ref, c1_ref,
                w2_ref, c2_ref, w3_ref, c3_ref, hw_ref, hb_ref, o_ref):
    f32 = jnp.float32
    h2 = (aggp_ref[0] + aggp_ref[1]
          + jnp.dot(h_ref[...], root_ref[...], preferred_element_type=f32)
          + bias_ref[...])
    ohT = (bi_ref[...] == lax.broadcasted_iota(jnp.int32, (NG, NPAD), 0)
           ).astype(f32)
    sums = jnp.dot(ohT, h2, preferred_element_type=f32)
    cnts = jnp.dot(ohT, jnp.ones_like(h2), preferred_element_type=f32)
    g = sums / jnp.maximum(cnts, 1.0)
    g = jnp.maximum(jnp.dot(g, w1_ref[...], preferred_element_type=f32)
                    + c1_ref[...], 0.0)
    g = jnp.maximum(jnp.dot(g, w2_ref[...], preferred_element_type=f32)
                    + c2_ref[...], 0.0)
    g = jnp.maximum(jnp.dot(g, w3_ref[...], preferred_element_type=f32)
                    + c3_ref[...], 0.0)
    o_ref[...] = (jnp.dot(g, hw_ref[...], preferred_element_type=f32)
                  + hb_ref[...])


def _tc_final(aggp, h, root, bias, bip, w1, c1, w2, c2, w3, c3, hw, hb):
    return pl.pallas_call(
        _final_body,
        out_shape=jax.ShapeDtypeStruct((NG, 8), jnp.float32),
        name="tc_final",
    )(aggp, h, root, bias.reshape(1, CI), bip,
      w1, c1.reshape(1, CI), w2, c2.reshape(1, CI), w3, c3.reshape(1, CI),
      hw, hb.reshape(1, 8))


def kernel(x, edge_index, edge_attr, batch_idx,
           e1_W1, e1_b1, e1_W2, e1_b2, root1, bias1,
           e2_W1, e2_b1, e2_W2, e2_b2, root2, bias2,
           nn_W1, nn_b1, nn_W2, nn_b2, nn_W3, nn_b3,
           head_W, head_b):
    f32 = jnp.float32
    pad = CPT - CPT_R
    src = edge_index[0].reshape(NT, CPT_R)
    dst = edge_index[1].reshape(NT, CPT_R)
    srcp = jnp.pad(src, ((0, 0), (0, pad)),
                   constant_values=DUMMY).reshape(NT, NCHK, CHK)
    dstp = jnp.pad(dst, ((0, 0), (0, pad)),
                   constant_values=DUMMY).reshape(NT, NCHK, CHK)
    eap = jnp.pad(edge_attr.reshape(NT, CPT_R, CI),
                  ((0, 0), (0, pad), (0, 0))).reshape(EPAD, CI)
    xpad = jnp.pad(x, ((0, NPAD - N), (0, 0)))
    bip = jnp.pad(batch_idx, (0, NPAD - N),
                  constant_values=NG).reshape(1, NPAD)
    zN = jnp.zeros((NPAD, CI), f32)
    jj = jnp.arange(CW)
    Rm = (jj[None, :] // CI == jnp.arange(CI)[:, None]).astype(f32)
    Sm = (jj[:, None] % CI == jnp.arange(CI)[None, :]).astype(f32)

    sc_gather, sc_scatter = _sc_kernels()
    xs = sc_gather(xpad, srcp).reshape(EPAD, CI)
    msg1 = _tc_edge(eap, xs, e1_W1, e1_b1.reshape(1, CEH), e1_W2,
                    e1_b2.reshape(1, CW), Rm, Sm)
    agg1 = sc_scatter(msg1.reshape(NT, NCHK, CHK, CI), dstp, zN)
    h = _tc_node(agg1, xpad, root1, bias1)
    hs = sc_gather(h, srcp).reshape(EPAD, CI)
    msg2 = _tc_edge(eap, hs, e2_W1, e2_b1.reshape(1, CEH), e2_W2,
                    e2_b2.reshape(1, CW), Rm, Sm)
    agg2 = sc_scatter(msg2.reshape(NT, NCHK, CHK, CI), dstp, zN)
    return _tc_final(agg2, h, root2, bias2, bip,
                     nn_W1, nn_b1, nn_W2, nn_b2, nn_W3, nn_b3,
                     head_W, head_b)


# speedup vs baseline: 3.4559x; 1.0041x over previous
import functools

import jax
import jax.numpy as jnp
from jax import lax
from jax.experimental import pallas as pl
from jax.experimental.pallas import tpu as pltpu
from jax.experimental.pallas import tpu_sc as plsc

N = 10000
E = 160000
NG = 64
CI = 16
CEH = 32
CW = 256

NT = 32
CPT_R = E // NT
CHK = 128
NCHK = 40
CPT = NCHK * CHK
EPAD = NT * CPT
NPAD = 10016
DUMMY = N
NSUB = 16
RPS = NPAD // NSUB


@functools.lru_cache(maxsize=None)
def _sc_kernels():
    mesh = plsc.VectorSubcoreMesh(core_axis_name="c", subcore_axis_name="s")
    params = pltpu.CompilerParams(use_tc_tiling_on_sc=False)

    @functools.partial(
        pl.kernel,
        out_type=jax.ShapeDtypeStruct((EPAD, CI), jnp.float32),
        mesh=mesh,
        scratch_types=[
            pltpu.VMEM((NCHK, CHK), jnp.int32),
            pltpu.VMEM((CPT, CI), jnp.float32),
            pltpu.VMEM_SHARED((NPAD, CI), jnp.float32),
        ],
        compiler_params=params,
        name="sc_gather",
    )
    def _sc_gather(table_hbm, idx_hbm, out_hbm, idx_v, rows_v, tab_sh):
        s = lax.axis_index("s")
        wid = s * 2 + lax.axis_index("c")
        pltpu.sync_copy(table_hbm.at[pl.ds(s * RPS, RPS)],
                        tab_sh.at[pl.ds(s * RPS, RPS)])
        pltpu.sync_copy(idx_hbm.at[wid], idx_v)
        plsc.subcore_barrier()

        def body(k, carry):
            pltpu.sync_copy(tab_sh.at[idx_v.at[k]],
                            rows_v.at[pl.ds(k * CHK, CHK)])
            return carry

        lax.fori_loop(0, NCHK, body, 0)
        pltpu.sync_copy(rows_v, out_hbm.at[pl.ds(wid * CPT, CPT)])

    @functools.partial(
        pl.kernel,
        out_type=(jax.ShapeDtypeStruct((NPAD, CI), jnp.float32),
                  jax.ShapeDtypeStruct((EPAD, CI), jnp.float32)),
        mesh=mesh,
        scratch_types=[
            pltpu.VMEM((NCHK, CHK), jnp.int32),
            pltpu.VMEM((CPT, CI), jnp.float32),
            pltpu.VMEM((RPS, CI), jnp.float32),
            pltpu.VMEM((RPS, CI), jnp.float32),
            pltpu.VMEM((RPS, CI), jnp.float32),
            pltpu.VMEM_SHARED((NPAD, CI), jnp.float32),
        ],
        compiler_params=params,
        name="sc_node_gather",
    )
    def _sc_node_gather(agg_hbm, xrb_hbm, idx_hbm, h_hbm, out_hbm,
                        idx_v, rows_v, a_v, b_v, c_v, tab_sh):
        s = lax.axis_index("s")
        wid = s * 2 + lax.axis_index("c")
        pltpu.sync_copy(agg_hbm.at[0, pl.ds(s * RPS, RPS)], a_v)
        pltpu.sync_copy(agg_hbm.at[1, pl.ds(s * RPS, RPS)], b_v)
        pltpu.sync_copy(xrb_hbm.at[pl.ds(s * RPS, RPS)], c_v)
        pltpu.sync_copy(idx_hbm.at[wid], idx_v)

        def rowbody(i, carry):
            a_v[i] = jnp.maximum(a_v[i] + b_v[i] + c_v[i], 0.0)
            return carry

        lax.fori_loop(0, RPS, rowbody, 0)
        pltpu.sync_copy(a_v, tab_sh.at[pl.ds(s * RPS, RPS)])
        pltpu.sync_copy(a_v, h_hbm.at[pl.ds(s * RPS, RPS)])
        plsc.subcore_barrier()

        def body(k, carry):
            pltpu.sync_copy(tab_sh.at[idx_v.at[k]],
                            rows_v.at[pl.ds(k * CHK, CHK)])
            return carry

        lax.fori_loop(0, NCHK, body, 0)
        pltpu.sync_copy(rows_v, out_hbm.at[pl.ds(wid * CPT, CPT)])

    @functools.partial(
        pl.kernel,
        out_type=jax.ShapeDtypeStruct((2, NPAD, CI), jnp.float32),
        mesh=mesh,
        scratch_types=[
            pltpu.VMEM((NCHK, CHK), jnp.int32),
            pltpu.VMEM((CPT, CI), jnp.float32),
            pltpu.VMEM_SHARED((NPAD, CI), jnp.float32),
        ],
        compiler_params=params,
        name="sc_scatter",
    )
    def _sc_scatter(msg_hbm, idx_hbm, zeros_hbm, out_hbm, idx_v, rows_v,
                    acc_sh):
        c = lax.axis_index("c")
        s = lax.axis_index("s")
        wid = s * 2 + c
        pltpu.sync_copy(zeros_hbm.at[pl.ds(s * RPS, RPS)],
                        acc_sh.at[pl.ds(s * RPS, RPS)])
        pltpu.sync_copy(idx_hbm.at[wid], idx_v)
        pltpu.sync_copy(msg_hbm.at[pl.ds(wid * CPT, CPT)], rows_v)
        plsc.subcore_barrier()

        def body(k, carry):
            pltpu.sync_copy(rows_v.at[pl.ds(k * CHK, CHK)],
                            acc_sh.at[idx_v.at[k]], add=True)
            return carry

        lax.fori_loop(0, NCHK, body, 0)
        plsc.subcore_barrier()
        pltpu.sync_copy(acc_sh.at[pl.ds(s * RPS, RPS)],
                        out_hbm.at[c, pl.ds(s * RPS, RPS)])

    return _sc_gather, _sc_node_gather, _sc_scatter


BLK = 2048


def _edge_body(ea_ref, xs_ref, w1_ref, b1_ref, w2_ref, b2_ref, r_ref, s_ref,
               tab_ref, root_ref, bias_ref, o_ref, trb_ref):
    f32 = jnp.float32
    i = pl.program_id(0)

    @pl.when(i == 0)
    def _():
        trb_ref[...] = (jnp.dot(tab_ref[...], root_ref[...],
                                preferred_element_type=f32) + bias_ref[...])

    h = jnp.maximum(
        jnp.dot(ea_ref[...], w1_ref[...], preferred_element_type=f32)
        + b1_ref[...], 0.0)
    w = jnp.maximum(
        jnp.dot(h, w2_ref[...], preferred_element_type=f32) + b2_ref[...],
        0.0)
    xe = jnp.dot(xs_ref[...], r_ref[...], preferred_element_type=f32)
    o_ref[...] = jnp.dot(xe * w, s_ref[...], preferred_element_type=f32)


def _tc_edge(ea, xs, W1, b1, W2, b2, Rm, Sm, tab, root, bias):
    return pl.pallas_call(
        _edge_body,
        grid=(EPAD // BLK,),
        in_specs=[
            pl.BlockSpec((BLK, CI), lambda i: (i, 0)),
            pl.BlockSpec((BLK, CI), lambda i: (i, 0)),
            pl.BlockSpec((CI, CEH), lambda i: (0, 0)),
            pl.BlockSpec((1, CEH), lambda i: (0, 0)),
            pl.BlockSpec((CEH, CW), lambda i: (0, 0)),
            pl.BlockSpec((1, CW), lambda i: (0, 0)),
            pl.BlockSpec((CI, CW), lambda i: (0, 0)),
            pl.BlockSpec((CW, CI), lambda i: (0, 0)),
            pl.BlockSpec((NPAD, CI), lambda i: (0, 0)),
            pl.BlockSpec((CI, CI), lambda i: (0, 0)),
            pl.BlockSpec((1, CI), lambda i: (0, 0)),
        ],
        out_specs=[
            pl.BlockSpec((BLK, CI), lambda i: (i, 0)),
            pl.BlockSpec((NPAD, CI), lambda i: (0, 0)),
        ],
        out_shape=[
            jax.ShapeDtypeStruct((EPAD, CI), jnp.float32),
            jax.ShapeDtypeStruct((NPAD, CI), jnp.float32),
        ],
        name="tc_edge",
    )(ea, xs, W1, b1, W2, b2, Rm, Sm, tab, root, bias.reshape(1, CI))


def _final_body(aggp_ref, hrb_ref, bi_ref, w1_ref, c1_ref,
                w2_ref, c2_ref, w3_ref, c3_ref, hw_ref, hb_ref, o_ref):
    f32 = jnp.float32
    h2 = aggp_ref[0] + aggp_ref[1] + hrb_ref[...]
    ohT = (bi_ref[...] == lax.broadcasted_iota(jnp.int32, (NG, NPAD), 0)
           ).astype(f32)
    sums = jnp.dot(ohT, h2, preferred_element_type=f32)
    cnts = jnp.dot(ohT, jnp.ones_like(h2), preferred_element_type=f32)
    g = sums / jnp.maximum(cnts, 1.0)
    g = jnp.maximum(jnp.dot(g, w1_ref[...], preferred_element_type=f32)
                    + c1_ref[...], 0.0)
    g = jnp.maximum(jnp.dot(g, w2_ref[...], preferred_element_type=f32)
                    + c2_ref[...], 0.0)
    g = jnp.maximum(jnp.dot(g, w3_ref[...], preferred_element_type=f32)
                    + c3_ref[...], 0.0)
    o_ref[...] = (jnp.dot(g, hw_ref[...], preferred_element_type=f32)
                  + hb_ref[...])


def _tc_final(aggp, hrb, bip, w1, c1, w2, c2, w3, c3, hw, hb):
    return pl.pallas_call(
        _final_body,
        out_shape=jax.ShapeDtypeStruct((NG, 8), jnp.float32),
        name="tc_final",
    )(aggp, hrb, bip,
      w1, c1.reshape(1, CI), w2, c2.reshape(1, CI), w3, c3.reshape(1, CI),
      hw, hb.reshape(1, 8))


def kernel(x, edge_index, edge_attr, batch_idx,
           e1_W1, e1_b1, e1_W2, e1_b2, root1, bias1,
           e2_W1, e2_b1, e2_W2, e2_b2, root2, bias2,
           nn_W1, nn_b1, nn_W2, nn_b2, nn_W3, nn_b3,
           head_W, head_b):
    f32 = jnp.float32
    pad = CPT - CPT_R
    src = edge_index[0].reshape(NT, CPT_R)
    dst = edge_index[1].reshape(NT, CPT_R)
    srcp = jnp.pad(src, ((0, 0), (0, pad)),
                   constant_values=DUMMY).reshape(NT, NCHK, CHK)
    dstp = jnp.pad(dst, ((0, 0), (0, pad)),
                   constant_values=DUMMY).reshape(NT, NCHK, CHK)
    eap = jnp.pad(edge_attr.reshape(NT, CPT_R, CI),
                  ((0, 0), (0, pad), (0, 0))).reshape(EPAD, CI)
    xpad = jnp.pad(x, ((0, NPAD - N), (0, 0)))
    bip = jnp.pad(batch_idx, (0, NPAD - N),
                  constant_values=NG).reshape(1, NPAD)
    zN = jnp.zeros((NPAD, CI), f32)
    jj = jnp.arange(CW)
    Rm = (jj[None, :] // CI == jnp.arange(CI)[:, None]).astype(f32)
    Sm = (jj[:, None] % CI == jnp.arange(CI)[None, :]).astype(f32)

    sc_gather, sc_node_gather, sc_scatter = _sc_kernels()
    xs = sc_gather(xpad, srcp)
    msg1, xrb = _tc_edge(eap, xs, e1_W1, e1_b1.reshape(1, CEH), e1_W2,
                         e1_b2.reshape(1, CW), Rm, Sm, xpad, root1, bias1)
    agg1 = sc_scatter(msg1, dstp, zN)
    h, hs = sc_node_gather(agg1, xrb, srcp)
    msg2, hrb = _tc_edge(eap, hs, e2_W1, e2_b1.reshape(1, CEH), e2_W2,
                         e2_b2.reshape(1, CW), Rm, Sm, h, root2, bias2)
    agg2 = sc_scatter(msg2, dstp, zN)
    return _tc_final(agg2, hrb, bip,
                     nn_W1, nn_b1, nn_W2, nn_b2, nn_W3, nn_b3,
                     head_W, head_b)


# speedup vs baseline: 3.9121x; 1.1320x over previous
import functools

import jax
import jax.numpy as jnp
from jax import lax
from jax.experimental import pallas as pl
from jax.experimental.pallas import tpu as pltpu
from jax.experimental.pallas import tpu_sc as plsc

N = 10000
E = 160000
NG = 64
CI = 16
CEH = 32
CW = 256

NT = 32
CPT = E // NT
CHK = 125
NCHK = 40
NPAD = 10016
NSUB = 16
RPS = NPAD // NSUB


@functools.lru_cache(maxsize=None)
def _sc_kernels():
    mesh = plsc.VectorSubcoreMesh(core_axis_name="c", subcore_axis_name="s")
    params = pltpu.CompilerParams(use_tc_tiling_on_sc=False)

    @functools.partial(
        pl.kernel,
        out_type=jax.ShapeDtypeStruct((E, CI), jnp.float32),
        mesh=mesh,
        scratch_types=[
            pltpu.VMEM((NCHK, CHK), jnp.int32),
            pltpu.VMEM((CPT, CI), jnp.float32),
            pltpu.VMEM_SHARED((NPAD, CI), jnp.float32),
        ],
        compiler_params=params,
        name="sc_gather",
    )
    def _sc_gather(table_hbm, idx_hbm, out_hbm, idx_v, rows_v, tab_sh):
        s = lax.axis_index("s")
        wid = s * 2 + lax.axis_index("c")
        pltpu.sync_copy(table_hbm.at[pl.ds(s * RPS, RPS)],
                        tab_sh.at[pl.ds(s * RPS, RPS)])
        pltpu.sync_copy(idx_hbm.at[wid], idx_v)
        plsc.subcore_barrier()

        def body(k, carry):
            pltpu.sync_copy(tab_sh.at[idx_v.at[k]],
                            rows_v.at[pl.ds(k * CHK, CHK)])
            return carry

        lax.fori_loop(0, NCHK, body, 0)
        pltpu.sync_copy(rows_v, out_hbm.at[pl.ds(wid * CPT, CPT)])

    @functools.partial(
        pl.kernel,
        out_type=(jax.ShapeDtypeStruct((NPAD, CI), jnp.float32),
                  jax.ShapeDtypeStruct((E, CI), jnp.float32)),
        mesh=mesh,
        scratch_types=[
            pltpu.VMEM((NCHK, CHK), jnp.int32),
            pltpu.VMEM((CPT, CI), jnp.float32),
            pltpu.VMEM((RPS, CI), jnp.float32),
            pltpu.VMEM((RPS, CI), jnp.float32),
            pltpu.VMEM((RPS, CI), jnp.float32),
            pltpu.VMEM_SHARED((NPAD, CI), jnp.float32),
        ],
        compiler_params=params,
        name="sc_node_gather",
    )
    def _sc_node_gather(agg_hbm, xrb_hbm, idx_hbm, h_hbm, out_hbm,
                        idx_v, rows_v, a_v, b_v, c_v, tab_sh):
        s = lax.axis_index("s")
        wid = s * 2 + lax.axis_index("c")
        pltpu.sync_copy(agg_hbm.at[0, pl.ds(s * RPS, RPS)], a_v)
        pltpu.sync_copy(agg_hbm.at[1, pl.ds(s * RPS, RPS)], b_v)
        pltpu.sync_copy(xrb_hbm.at[pl.ds(s * RPS, RPS)], c_v)
        pltpu.sync_copy(idx_hbm.at[wid], idx_v)

        def rowbody(i, carry):
            a_v[i] = jnp.maximum(a_v[i] + b_v[i] + c_v[i], 0.0)
            return carry

        lax.fori_loop(0, RPS, rowbody, 0)
        pltpu.sync_copy(a_v, tab_sh.at[pl.ds(s * RPS, RPS)])
        pltpu.sync_copy(a_v, h_hbm.at[pl.ds(s * RPS, RPS)])
        plsc.subcore_barrier()

        def body(k, carry):
            pltpu.sync_copy(tab_sh.at[idx_v.at[k]],
                            rows_v.at[pl.ds(k * CHK, CHK)])
            return carry

        lax.fori_loop(0, NCHK, body, 0)
        pltpu.sync_copy(rows_v, out_hbm.at[pl.ds(wid * CPT, CPT)])

    @functools.partial(
        pl.kernel,
        out_type=jax.ShapeDtypeStruct((2, NPAD, CI), jnp.float32),
        mesh=mesh,
        scratch_types=[
            pltpu.VMEM((NCHK, CHK), jnp.int32),
            pltpu.VMEM((CPT, CI), jnp.float32),
            pltpu.VMEM_SHARED((NPAD, CI), jnp.float32),
        ],
        compiler_params=params,
        name="sc_scatter",
    )
    def _sc_scatter(msg_hbm, idx_hbm, zeros_hbm, out_hbm, idx_v, rows_v,
                    acc_sh):
        c = lax.axis_index("c")
        s = lax.axis_index("s")
        wid = s * 2 + c
        pltpu.sync_copy(zeros_hbm.at[pl.ds(s * RPS, RPS)],
                        acc_sh.at[pl.ds(s * RPS, RPS)])
        pltpu.sync_copy(idx_hbm.at[wid], idx_v)
        pltpu.sync_copy(msg_hbm.at[pl.ds(wid * CPT, CPT)], rows_v)
        plsc.subcore_barrier()

        def body(k, carry):
            pltpu.sync_copy(rows_v.at[pl.ds(k * CHK, CHK)],
                            acc_sh.at[idx_v.at[k]], add=True)
            return carry

        lax.fori_loop(0, NCHK, body, 0)
        plsc.subcore_barrier()
        pltpu.sync_copy(acc_sh.at[pl.ds(s * RPS, RPS)],
                        out_hbm.at[c, pl.ds(s * RPS, RPS)])

    return _sc_gather, _sc_node_gather, _sc_scatter


BLK = 4000


def _edge_body(ea_ref, xs_ref, w1_ref, b1_ref, w2_ref, b2_ref, r_ref, s_ref,
               tab_ref, root_ref, bias_ref, o_ref, trb_ref):
    f32 = jnp.float32
    i = pl.program_id(0)

    @pl.when(i == 0)
    def _():
        trb_ref[...] = (jnp.dot(tab_ref[...], root_ref[...],
                                preferred_element_type=f32) + bias_ref[...])

    h = jnp.maximum(
        jnp.dot(ea_ref[...], w1_ref[...], preferred_element_type=f32)
        + b1_ref[...], 0.0)
    w = jnp.maximum(
        jnp.dot(h, w2_ref[...], preferred_element_type=f32) + b2_ref[...],
        0.0)
    xe = jnp.dot(xs_ref[...], r_ref[...], preferred_element_type=f32)
    o_ref[...] = jnp.dot(xe * w, s_ref[...], preferred_element_type=f32)


def _tc_edge(ea, xs, W1, b1, W2, b2, Rm, Sm, tab, root, bias):
    return pl.pallas_call(
        _edge_body,
        grid=(E // BLK,),
        in_specs=[
            pl.BlockSpec((BLK, CI), lambda i: (i, 0)),
            pl.BlockSpec((BLK, CI), lambda i: (i, 0)),
            pl.BlockSpec((CI, CEH), lambda i: (0, 0)),
            pl.BlockSpec((1, CEH), lambda i: (0, 0)),
            pl.BlockSpec((CEH, CW), lambda i: (0, 0)),
            pl.BlockSpec((1, CW), lambda i: (0, 0)),
            pl.BlockSpec((CI, CW), lambda i: (0, 0)),
            pl.BlockSpec((CW, CI), lambda i: (0, 0)),
            pl.BlockSpec((NPAD, CI), lambda i: (0, 0)),
            pl.BlockSpec((CI, CI), lambda i: (0, 0)),
            pl.BlockSpec((1, CI), lambda i: (0, 0)),
        ],
        out_specs=[
            pl.BlockSpec((BLK, CI), lambda i: (i, 0)),
            pl.BlockSpec((NPAD, CI), lambda i: (0, 0)),
        ],
        out_shape=[
            jax.ShapeDtypeStruct((E, CI), jnp.float32),
            jax.ShapeDtypeStruct((NPAD, CI), jnp.float32),
        ],
        name="tc_edge",
    )(ea, xs, W1, b1, W2, b2, Rm, Sm, tab, root, bias.reshape(1, CI))


def _final_body(aggp_ref, hrb_ref, bi_ref, w1_ref, c1_ref,
                w2_ref, c2_ref, w3_ref, c3_ref, hw_ref, hb_ref, o_ref):
    f32 = jnp.float32
    h2 = aggp_ref[0] + aggp_ref[1] + hrb_ref[...]
    ohT = (bi_ref[...] == lax.broadcasted_iota(jnp.int32, (NG, NPAD), 0)
           ).astype(f32)
    sums = jnp.dot(ohT, h2, preferred_element_type=f32)
    cnts = jnp.dot(ohT, jnp.ones_like(h2), preferred_element_type=f32)
    g = sums / jnp.maximum(cnts, 1.0)
    g = jnp.maximum(jnp.dot(g, w1_ref[...], preferred_element_type=f32)
                    + c1_ref[...], 0.0)
    g = jnp.maximum(jnp.dot(g, w2_ref[...], preferred_element_type=f32)
                    + c2_ref[...], 0.0)
    g = jnp.maximum(jnp.dot(g, w3_ref[...], preferred_element_type=f32)
                    + c3_ref[...], 0.0)
    o_ref[...] = (jnp.dot(g, hw_ref[...], preferred_element_type=f32)
                  + hb_ref[...])


def _tc_final(aggp, hrb, bip, w1, c1, w2, c2, w3, c3, hw, hb):
    return pl.pallas_call(
        _final_body,
        out_shape=jax.ShapeDtypeStruct((NG, 8), jnp.float32),
        name="tc_final",
    )(aggp, hrb, bip,
      w1, c1.reshape(1, CI), w2, c2.reshape(1, CI), w3, c3.reshape(1, CI),
      hw, hb.reshape(1, 8))


def kernel(x, edge_index, edge_attr, batch_idx,
           e1_W1, e1_b1, e1_W2, e1_b2, root1, bias1,
           e2_W1, e2_b1, e2_W2, e2_b2, root2, bias2,
           nn_W1, nn_b1, nn_W2, nn_b2, nn_W3, nn_b3,
           head_W, head_b):
    f32 = jnp.float32
    srcp = edge_index[0].reshape(NT, NCHK, CHK)
    dstp = edge_index[1].reshape(NT, NCHK, CHK)
    eap = edge_attr
    xpad = jnp.pad(x, ((0, NPAD - N), (0, 0)))
    bip = jnp.pad(batch_idx, (0, NPAD - N),
                  constant_values=NG).reshape(1, NPAD)
    zN = jnp.zeros((NPAD, CI), f32)
    jj = jnp.arange(CW)
    Rm = (jj[None, :] // CI == jnp.arange(CI)[:, None]).astype(f32)
    Sm = (jj[:, None] % CI == jnp.arange(CI)[None, :]).astype(f32)

    sc_gather, sc_node_gather, sc_scatter = _sc_kernels()
    xs = sc_gather(xpad, srcp)
    msg1, xrb = _tc_edge(eap, xs, e1_W1, e1_b1.reshape(1, CEH), e1_W2,
                         e1_b2.reshape(1, CW), Rm, Sm, xpad, root1, bias1)
    agg1 = sc_scatter(msg1, dstp, zN)
    h, hs = sc_node_gather(agg1, xrb, srcp)
    msg2, hrb = _tc_edge(eap, hs, e2_W1, e2_b1.reshape(1, CEH), e2_W2,
                         e2_b2.reshape(1, CW), Rm, Sm, h, root2, bias2)
    agg2 = sc_scatter(msg2, dstp, zN)
    return _tc_final(agg2, hrb, bip,
                     nn_W1, nn_b1, nn_W2, nn_b2, nn_W3, nn_b3,
                     head_W, head_b)
